# baseline (speedup 1.0000x reference)
import functools

import jax
import jax.numpy as jnp
from jax import lax
from jax.experimental import pallas as pl
from jax.experimental.pallas import tpu as pltpu
from jax.experimental.pallas import tpu_sc as plsc

N_NODES = 10000
N_EDGES = 160000

_info = plsc.get_sparse_core_info()
_NC, _NS, _L = _info.num_cores, _info.num_subcores, _info.num_lanes
_NW = _NC * _NS
NPS = 320
NPAD = _NW * NPS


def _mm2_body(x_ref, wl_ref, wr_ref, xl_ref, xr_ref):
    x = x_ref[...]
    xl_ref[...] = jax.lax.dot_general(
        x, wl_ref[...], (((1,), (0,)), ((), ())),
        preferred_element_type=jnp.float32)
    xr_ref[...] = jax.lax.dot_general(
        x, wr_ref[...], (((1,), (0,)), ((), ())),
        preferred_element_type=jnp.float32)


def _mm2(x, wl, wr, block_n=1000):
    n, k = x.shape
    c = wl.shape[1]
    return pl.pallas_call(
        _mm2_body,
        grid=(n // block_n,),
        in_specs=[
            pl.BlockSpec((block_n, k), lambda i: (i, 0)),
            pl.BlockSpec((k, c), lambda i: (0, 0)),
            pl.BlockSpec((k, c), lambda i: (0, 0)),
        ],
        out_specs=[
            pl.BlockSpec((block_n, c), lambda i: (i, 0)),
            pl.BlockSpec((block_n, c), lambda i: (i, 0)),
        ],
        out_shape=[
            jax.ShapeDtypeStruct((n, c), jnp.float32),
            jax.ShapeDtypeStruct((n, c), jnp.float32),
        ],
    )(x, wl, wr)


def _relu_mm2_body(h_ref, b_ref, wl_ref, wr_ref, xl_ref, xr_ref):
    h = jnp.maximum(h_ref[...] + b_ref[...][None, :], 0.0)
    xl_ref[...] = jax.lax.dot_general(
        h, wl_ref[...], (((1,), (0,)), ((), ())),
        preferred_element_type=jnp.float32)
    xr_ref[...] = jax.lax.dot_general(
        h, wr_ref[...], (((1,), (0,)), ((), ())),
        preferred_element_type=jnp.float32)


def _relu_mm2(hpre, b, wl, wr, block_n=1000):
    n, k = hpre.shape
    c = wl.shape[1]
    return pl.pallas_call(
        _relu_mm2_body,
        grid=(n // block_n,),
        in_specs=[
            pl.BlockSpec((block_n, k), lambda i: (i, 0)),
            pl.BlockSpec((k,), lambda i: (0,)),
            pl.BlockSpec((k, c), lambda i: (0, 0)),
            pl.BlockSpec((k, c), lambda i: (0, 0)),
        ],
        out_specs=[
            pl.BlockSpec((block_n, c), lambda i: (i, 0)),
            pl.BlockSpec((block_n, c), lambda i: (i, 0)),
        ],
        out_shape=[
            jax.ShapeDtypeStruct((n, c), jnp.float32),
            jax.ShapeDtypeStruct((n, c), jnp.float32),
        ],
    )(hpre, b, wl, wr)


def _bias_sigmoid_body(h_ref, b_ref, o_ref):
    o_ref[...] = jax.nn.sigmoid(h_ref[...] + b_ref[...][None, :])


def _bias_sigmoid(h, b, block_n=1000):
    n, c = h.shape
    return pl.pallas_call(
        _bias_sigmoid_body,
        grid=(n // block_n,),
        in_specs=[
            pl.BlockSpec((block_n, c), lambda i: (i, 0)),
            pl.BlockSpec((c,), lambda i: (0,)),
        ],
        out_specs=pl.BlockSpec((block_n, c), lambda i: (i, 0)),
        out_shape=jax.ShapeDtypeStruct((n, c), jnp.float32),
    )(h, b)


def _sc_edge_kernel(cp, unroll=8):
    nck = cp // _L
    mesh = plsc.VectorSubcoreMesh(core_axis_name="c", subcore_axis_name="s")

    @functools.partial(
        pl.kernel, mesh=mesh,
        out_type=jax.ShapeDtypeStruct((NPAD, cp), jnp.float32),
        scratch_types=[
            pltpu.VMEM((48,), jnp.int32),
            pltpu.VMEM((2, 2, 16), jnp.int32),
            pltpu.VMEM((2, 16), jnp.float32),
            pltpu.VMEM((2, 16, cp), jnp.float32),
            pltpu.VMEM((2, 16, cp), jnp.float32),
            pltpu.VMEM((cp,), jnp.float32),
            pltpu.VMEM((cp,), jnp.float32),
            pltpu.VMEM((cp,), jnp.float32),
            pltpu.VMEM((cp,), jnp.float32),
            pltpu.SemaphoreType.DMA,
        ],
    )
    def k(xl_ref, xr_ref, edat_ref, a_ref, offs_ref, we_ref,
          att_ref, out_ref, offs_v, edat_v, av_v, rows_v, xrr_v, accv,
          rowv, wev, attv, sem):
        wid = lax.axis_index("s") * _NC + lax.axis_index("c")
        n0 = wid * NPS
        iota = lax.iota(jnp.int32, _L)

        gdn = lax.GatherDimensionNumbers(
            offset_dims=(), collapsed_slice_dims=(0,), start_index_map=(0,))

        def lane_sum_splat(v):
            for sh in (8, 4, 2, 1):
                perm = lax.gather(
                    v, (iota ^ sh)[:, None], gdn, slice_sizes=(1,),
                    mode=lax.GatherScatterMode.PROMISE_IN_BOUNDS)
                v = v + perm
            return v
        zero16 = jnp.zeros((_L,), jnp.float32)
        neg16 = jnp.full((_L,), -1e30, jnp.float32)

        pltpu.sync_copy(offs_ref, offs_v)
        pltpu.sync_copy(we_ref, wev)
        pltpu.sync_copy(att_ref, attv)

        e0 = offs_v[pl.ds(wid, 16)][0]
        e1 = offs_v[pl.ds(wid + 1, 16)][0]
        ba = (e0 >> 4) << 4
        nwin = (e1 - ba + 15) >> 4

        def zrow(j, _):
            @plsc.parallel_loop(0, nck, unroll=unroll)
            def _zck(ck):
                rows_v[0, j, pl.ds(ck * _L, _L)] = zero16
            return 0
        lax.fori_loop(0, 16, zrow, 0)

        def zdma(j, _):
            pltpu.sync_copy(rows_v.at[0],
                            out_ref.at[pl.ds(n0 + j * 16, 16), :])
            return 0
        lax.fori_loop(0, NPS // 16, zdma, 0)

        @plsc.parallel_loop(0, nck, unroll=unroll)
        def _zacc(ck):
            accv[pl.ds(ck * _L, _L)] = zero16

        def finalize(d_cur, s_v):
            inv = 1.0 / (s_v + 1e-16)
            @plsc.parallel_loop(0, nck, unroll=unroll)
            def _fck(ck):
                sl = pl.ds(ck * _L, _L)
                rowv[sl] = accv[sl] * inv
                accv[sl] = zero16
            pltpu.sync_copy(rowv, out_ref.at[d_cur])

        wa0 = ba >> 4
        one16 = jnp.ones((_L,), jnp.float32)
        onehots = [jnp.where(iota == i, 1.0, 0.0).astype(jnp.float32)
                   for i in range(16)]
        pltpu.sync_copy(edat_ref.at[wa0], edat_v.at[0])
        pltpu.sync_copy(a_ref.at[wa0], av_v.at[0])
        pltpu.async_copy(xl_ref.at[edat_v[0, 0, :]], rows_v.at[0], sem)
        pltpu.async_copy(xr_ref.at[edat_v[0, 1, :]], xrr_v.at[0], sem)

        def window(w, carry):
            d_cur, m_v, s_v = carry
            wb = pl.multiple_of(ba, 16) + w * 16
            cur = w & 1
            nxt = 1 - cur
            pltpu.make_async_copy(
                xl_ref.at[pl.ds(0, 16), :], rows_v.at[cur], sem).wait()
            pltpu.make_async_copy(
                xr_ref.at[pl.ds(0, 16), :], xrr_v.at[cur], sem).wait()
            pltpu.sync_copy(edat_ref.at[wa0 + w + 1], edat_v.at[nxt])
            pltpu.sync_copy(a_ref.at[wa0 + w + 1], av_v.at[nxt])
            pltpu.async_copy(
                xl_ref.at[edat_v[nxt, 0, :]], rows_v.at[nxt], sem)
            pltpu.async_copy(
                xr_ref.at[edat_v[nxt, 1, :]], xrr_v.at[nxt], sem)

            dvec = edat_v[cur, 1, :]
            avec = av_v[cur, :]
            a_sp = [lax.broadcast(avec[i], (_L,)) for i in range(16)]

            @plsc.parallel_loop(0, nck, unroll=2,
                                carry=tuple([zero16] * 16))
            def accs(ck, acc):
                sl = pl.ds(ck * _L, _L)
                we_c = wev[sl]
                att_c = attv[sl]
                new = []
                for i in range(16):
                    z = rows_v[cur, i, sl] + (xrr_v[cur, i, sl]
                                              + a_sp[i] * we_c)
                    new.append(acc[i] + att_c * jnp.maximum(z, 0.2 * z))
                return tuple(new)

            alphas = [lane_sum_splat(accs[i]) for i in range(16)]

            As = one16
            bvec = zero16
            for i in range(16):
                e_ok = jnp.logical_and(wb + i >= e0, wb + i < e1)
                d_i = dvec[i]
                is_new = jnp.logical_and(e_ok, d_i != d_cur)

                @pl.when(jnp.logical_and(is_new, d_cur >= 0))
                def _flush():
                    inv = 1.0 / (s_v + 1e-16)
                    a0 = As * inv
                    cv = bvec * inv
                    c_sp = [lax.broadcast(cv[j], (_L,)) for j in range(16)]

                    @plsc.parallel_loop(0, nck, unroll=unroll)
                    def _fl(ck):
                        sl = pl.ds(ck * _L, _L)
                        r = a0 * accv[sl]
                        for j in range(16):
                            r = r + c_sp[j] * rows_v[cur, j, sl]
                        rowv[sl] = r
                    pltpu.sync_copy(rowv, out_ref.at[d_cur])

                As = jnp.where(is_new, zero16, As)
                bvec = jnp.where(is_new, zero16, bvec)
                d_cur = jnp.where(is_new, d_i, d_cur)
                m0 = jnp.where(is_new, neg16, m_v)
                s0 = jnp.where(is_new, zero16, s_v)

                alpha = alphas[i]
                new_m = jnp.maximum(m0, alpha)
                scale = jnp.where(e_ok, jnp.exp(m0 - new_m), one16)
                p = jnp.where(e_ok, jnp.exp(alpha - new_m), zero16)
                As = As * scale
                bvec = bvec * scale + p * onehots[i]
                s_v = s0 * scale + p
                m_v = jnp.where(e_ok, new_m, m0)

            b_sp = [lax.broadcast(bvec[j], (_L,)) for j in range(16)]

            @plsc.parallel_loop(0, nck, unroll=unroll)
            def _upd(ck):
                sl = pl.ds(ck * _L, _L)
                r = As * accv[sl]
                for j in range(16):
                    r = r + b_sp[j] * rows_v[cur, j, sl]
                accv[sl] = r

            return (d_cur, m_v, s_v)

        d_cur, m_v, s_v = lax.fori_loop(
            0, nwin, window,
            (jnp.int32(-1), neg16, jnp.zeros((_L,), jnp.float32)))

        pltpu.make_async_copy(
            xl_ref.at[pl.ds(0, 16), :], rows_v.at[0], sem).wait()
        pltpu.make_async_copy(
            xr_ref.at[pl.ds(0, 16), :], xrr_v.at[0], sem).wait()

        @pl.when(d_cur >= 0)
        def _fin_tail():
            finalize(d_cur, s_v)

    return k


def _pad_cols(m, cp):
    c = m.shape[-1]
    if c == cp:
        return m
    pad = [(0, 0)] * (m.ndim - 1) + [(0, cp - c)]
    return jnp.pad(m, pad)


def kernel(x, edge_index, edge_attr, Wl1, Wr1, We1, att1, b1,
           Wl2, Wr2, We2, att2, b2):
    src = edge_index[0].astype(jnp.int32)
    dst = edge_index[1].astype(jnp.int32)
    a = edge_attr[:, 0]

    perm = jnp.argsort(dst)
    dst_s = dst[perm]
    src_s = src[perm]
    a_s = a[perm]
    bnd = jnp.searchsorted(
        dst_s, jnp.arange(0, NPAD + 1, NPS, dtype=jnp.int32)).astype(jnp.int32)
    offs = jnp.pad(bnd, (0, 48 - bnd.shape[0]))
    src_p = jnp.pad(src_s, (0, 32))
    dst_p = jnp.pad(dst_s, (0, 32))
    a_p = jnp.pad(a_s, (0, 32))
    edat = jnp.stack([src_p, dst_p], axis=0)
    edat = edat.reshape(2, -1, 16).transpose(1, 0, 2)
    aw = a_p.reshape(-1, 16)

    cp1 = 1024
    cp2 = 128

    xl1, xr1 = _mm2(x, _pad_cols(Wl1, cp1), _pad_cols(Wr1, cp1))
    we1 = _pad_cols(We1[0][None, :], cp1)[0]
    at1 = _pad_cols(att1[None, :], cp1)[0]
    h1 = _sc_edge_kernel(cp1)(xl1, xr1, edat, aw, offs, we1, at1)

    b1p = _pad_cols(b1[None, :], cp1)[0]
    wl2 = jnp.pad(Wl2, ((0, cp1 - Wl2.shape[0]), (0, 0)))
    wr2 = jnp.pad(Wr2, ((0, cp1 - Wr2.shape[0]), (0, 0)))
    xl2, xr2 = _relu_mm2(h1[:N_NODES], b1p, wl2, wr2)
    h2 = _sc_edge_kernel(cp2)(xl2, xr2, edat, aw, offs, We2[0], att2)

    return _bias_sigmoid(h2[:N_NODES], b2)


# speedup vs baseline: 1.0284x; 1.0284x over previous
import functools

import jax
import jax.numpy as jnp
from jax import lax
from jax.experimental import pallas as pl
from jax.experimental.pallas import tpu as pltpu
from jax.experimental.pallas import tpu_sc as plsc

N_NODES = 10000
N_EDGES = 160000

_info = plsc.get_sparse_core_info()
_NC, _NS, _L = _info.num_cores, _info.num_subcores, _info.num_lanes
_NW = _NC * _NS
NPS = 320
NPAD = _NW * NPS


def _mm2_body(x_ref, wl_ref, wr_ref, xl_ref, xr_ref):
    x = x_ref[...]
    xl_ref[...] = jax.lax.dot_general(
        x, wl_ref[...], (((1,), (0,)), ((), ())),
        preferred_element_type=jnp.float32)
    xr_ref[...] = jax.lax.dot_general(
        x, wr_ref[...], (((1,), (0,)), ((), ())),
        preferred_element_type=jnp.float32)


def _mm2(x, wl, wr, block_n=1000):
    n, k = x.shape
    c = wl.shape[1]
    return pl.pallas_call(
        _mm2_body,
        grid=(n // block_n,),
        in_specs=[
            pl.BlockSpec((block_n, k), lambda i: (i, 0)),
            pl.BlockSpec((k, c), lambda i: (0, 0)),
            pl.BlockSpec((k, c), lambda i: (0, 0)),
        ],
        out_specs=[
            pl.BlockSpec((block_n, c), lambda i: (i, 0)),
            pl.BlockSpec((block_n, c), lambda i: (i, 0)),
        ],
        out_shape=[
            jax.ShapeDtypeStruct((n, c), jnp.float32),
            jax.ShapeDtypeStruct((n, c), jnp.float32),
        ],
    )(x, wl, wr)


def _relu_mm2_body(h_ref, b_ref, wl_ref, wr_ref, xl_ref, xr_ref):
    h = jnp.maximum(h_ref[...] + b_ref[...][None, :], 0.0)
    xl_ref[...] = jax.lax.dot_general(
        h, wl_ref[...], (((1,), (0,)), ((), ())),
        preferred_element_type=jnp.float32)
    xr_ref[...] = jax.lax.dot_general(
        h, wr_ref[...], (((1,), (0,)), ((), ())),
        preferred_element_type=jnp.float32)


def _relu_mm2(hpre, b, wl, wr, block_n=1000):
    n, k = hpre.shape
    c = wl.shape[1]
    return pl.pallas_call(
        _relu_mm2_body,
        grid=(n // block_n,),
        in_specs=[
            pl.BlockSpec((block_n, k), lambda i: (i, 0)),
            pl.BlockSpec((k,), lambda i: (0,)),
            pl.BlockSpec((k, c), lambda i: (0, 0)),
            pl.BlockSpec((k, c), lambda i: (0, 0)),
        ],
        out_specs=[
            pl.BlockSpec((block_n, c), lambda i: (i, 0)),
            pl.BlockSpec((block_n, c), lambda i: (i, 0)),
        ],
        out_shape=[
            jax.ShapeDtypeStruct((n, c), jnp.float32),
            jax.ShapeDtypeStruct((n, c), jnp.float32),
        ],
    )(hpre, b, wl, wr)


def _bias_sigmoid_body(h_ref, b_ref, o_ref):
    o_ref[...] = jax.nn.sigmoid(h_ref[...] + b_ref[...][None, :])


def _bias_sigmoid(h, b, block_n=1000):
    n, c = h.shape
    return pl.pallas_call(
        _bias_sigmoid_body,
        grid=(n // block_n,),
        in_specs=[
            pl.BlockSpec((block_n, c), lambda i: (i, 0)),
            pl.BlockSpec((c,), lambda i: (0,)),
        ],
        out_specs=pl.BlockSpec((block_n, c), lambda i: (i, 0)),
        out_shape=jax.ShapeDtypeStruct((n, c), jnp.float32),
    )(h, b)


def _sc_edge_kernel(cp, unroll=8):
    nck = cp // _L
    mesh = plsc.VectorSubcoreMesh(core_axis_name="c", subcore_axis_name="s")

    @functools.partial(
        pl.kernel, mesh=mesh,
        out_type=jax.ShapeDtypeStruct((NPAD, cp), jnp.float32),
        scratch_types=[
            pltpu.VMEM((48,), jnp.int32),
            pltpu.VMEM((2, 2, 16), jnp.int32),
            pltpu.VMEM((2, 16), jnp.float32),
            pltpu.VMEM((2, 16, cp), jnp.float32),
            pltpu.VMEM((cp,), jnp.float32),
            pltpu.VMEM((cp,), jnp.float32),
            pltpu.VMEM((cp,), jnp.float32),
            pltpu.VMEM((cp,), jnp.float32),
            pltpu.VMEM((cp,), jnp.float32),
            pltpu.SemaphoreType.DMA,
        ],
    )
    def k(xl_ref, xr_ref, edat_ref, a_ref, offs_ref, we_ref,
          att_ref, out_ref, offs_v, edat_v, av_v, rows_v, xrv, accv,
          rowv, wev, attv, sem):
        wid = lax.axis_index("s") * _NC + lax.axis_index("c")
        n0 = wid * NPS
        iota = lax.iota(jnp.int32, _L)

        gdn = lax.GatherDimensionNumbers(
            offset_dims=(), collapsed_slice_dims=(0,), start_index_map=(0,))

        def lane_sum_splat(v):
            for sh in (8, 4, 2, 1):
                perm = lax.gather(
                    v, (iota ^ sh)[:, None], gdn, slice_sizes=(1,),
                    mode=lax.GatherScatterMode.PROMISE_IN_BOUNDS)
                v = v + perm
            return v
        zero16 = jnp.zeros((_L,), jnp.float32)
        neg16 = jnp.full((_L,), -1e30, jnp.float32)

        pltpu.sync_copy(offs_ref, offs_v)
        pltpu.sync_copy(we_ref, wev)
        pltpu.sync_copy(att_ref, attv)

        e0 = offs_v[pl.ds(wid, 16)][0]
        e1 = offs_v[pl.ds(wid + 1, 16)][0]
        ba = (e0 >> 4) << 4
        nwin = (e1 - ba + 15) >> 4

        def zrow(j, _):
            @plsc.parallel_loop(0, nck, unroll=unroll)
            def _zck(ck):
                rows_v[0, j, pl.ds(ck * _L, _L)] = zero16
            return 0
        lax.fori_loop(0, 16, zrow, 0)

        def zdma(j, _):
            pltpu.sync_copy(rows_v.at[0],
                            out_ref.at[pl.ds(n0 + j * 16, 16), :])
            return 0
        lax.fori_loop(0, NPS // 16, zdma, 0)

        @plsc.parallel_loop(0, nck, unroll=unroll)
        def _zacc(ck):
            accv[pl.ds(ck * _L, _L)] = zero16

        def finalize(d_cur, s_v):
            inv = 1.0 / (s_v + 1e-16)
            @plsc.parallel_loop(0, nck, unroll=unroll)
            def _fck(ck):
                sl = pl.ds(ck * _L, _L)
                rowv[sl] = accv[sl] * inv
                accv[sl] = zero16
            pltpu.sync_copy(rowv, out_ref.at[d_cur])

        wa0 = ba >> 4
        pltpu.sync_copy(edat_ref.at[wa0], edat_v.at[0])
        pltpu.sync_copy(a_ref.at[wa0], av_v.at[0])
        pltpu.async_copy(xl_ref.at[edat_v[0, 0, :]], rows_v.at[0], sem)

        def window(w, carry):
            d_cur, m_v, s_v = carry
            wb = pl.multiple_of(ba, 16) + w * 16
            cur = w & 1
            nxt = 1 - cur
            pltpu.make_async_copy(
                xl_ref.at[pl.ds(0, 16), :], rows_v.at[cur], sem).wait()
            pltpu.sync_copy(edat_ref.at[wa0 + w + 1], edat_v.at[nxt])
            pltpu.sync_copy(a_ref.at[wa0 + w + 1], av_v.at[nxt])
            pltpu.async_copy(
                xl_ref.at[edat_v[nxt, 0, :]], rows_v.at[nxt], sem)

            dvec = edat_v[cur, 1, :]
            avec = av_v[cur, :]
            for i in range(16):
                e_ok = jnp.logical_and(wb + i >= e0, wb + i < e1)
                d_i = dvec[i]
                a_i = lax.broadcast(avec[i], (_L,))
                is_new = jnp.logical_and(e_ok, d_i != d_cur)

                @pl.when(jnp.logical_and(is_new, d_cur >= 0))
                def _fin():
                    finalize(d_cur, s_v)

                @pl.when(is_new)
                def _ld():
                    pltpu.sync_copy(xr_ref.at[d_i], xrv)

                d_cur = jnp.where(is_new, d_i, d_cur)
                m0 = jnp.where(is_new, neg16, m_v)
                s0 = jnp.where(is_new, jnp.zeros((_L,), jnp.float32), s_v)

                @plsc.parallel_loop(0, nck, unroll=unroll, carry=zero16)
                def acc16(ck, acc):
                    sl = pl.ds(ck * _L, _L)
                    z = rows_v[cur, i, sl] + xrv[sl] + a_i * wev[sl]
                    zl = jnp.maximum(z, 0.2 * z)
                    return acc + attv[sl] * zl

                alpha = lane_sum_splat(acc16)
                new_m = jnp.maximum(m0, alpha)
                scale = jnp.exp(m0 - new_m)
                p = jnp.exp(alpha - new_m)
                s_new = s0 * scale + p

                @pl.when(e_ok)
                def _acc():
                    @plsc.parallel_loop(0, nck, unroll=unroll)
                    def _uck(ck):
                        sl = pl.ds(ck * _L, _L)
                        accv[sl] = accv[sl] * scale + p * rows_v[cur, i, sl]

                m_v = jnp.where(e_ok, new_m, m0)
                s_v = jnp.where(e_ok, s_new, s0)
            return (d_cur, m_v, s_v)

        d_cur, m_v, s_v = lax.fori_loop(
            0, nwin, window,
            (jnp.int32(-1), neg16, jnp.zeros((_L,), jnp.float32)))

        pltpu.make_async_copy(
            xl_ref.at[pl.ds(0, 16), :], rows_v.at[0], sem).wait()

        @pl.when(d_cur >= 0)
        def _fin_tail():
            finalize(d_cur, s_v)

    return k


def _pad_cols(m, cp):
    c = m.shape[-1]
    if c == cp:
        return m
    pad = [(0, 0)] * (m.ndim - 1) + [(0, cp - c)]
    return jnp.pad(m, pad)


def kernel(x, edge_index, edge_attr, Wl1, Wr1, We1, att1, b1,
           Wl2, Wr2, We2, att2, b2):
    src = edge_index[0].astype(jnp.int32)
    dst = edge_index[1].astype(jnp.int32)
    a = edge_attr[:, 0]

    perm = jnp.argsort(dst)
    dst_s = dst[perm]
    src_s = src[perm]
    a_s = a[perm]
    bnd = jnp.searchsorted(
        dst_s, jnp.arange(0, NPAD + 1, NPS, dtype=jnp.int32)).astype(jnp.int32)
    offs = jnp.pad(bnd, (0, 48 - bnd.shape[0]))
    src_p = jnp.pad(src_s, (0, 32))
    dst_p = jnp.pad(dst_s, (0, 32))
    a_p = jnp.pad(a_s, (0, 32))
    edat = jnp.stack([src_p, dst_p], axis=0)
    edat = edat.reshape(2, -1, 16).transpose(1, 0, 2)
    aw = a_p.reshape(-1, 16)

    cp1 = 1024
    cp2 = 128

    xl1, xr1 = _mm2(x, _pad_cols(Wl1, cp1), _pad_cols(Wr1, cp1))
    we1 = _pad_cols(We1[0][None, :], cp1)[0]
    at1 = _pad_cols(att1[None, :], cp1)[0]
    h1 = _sc_edge_kernel(cp1)(xl1, xr1, edat, aw, offs, we1, at1)

    b1p = _pad_cols(b1[None, :], cp1)[0]
    wl2 = jnp.pad(Wl2, ((0, cp1 - Wl2.shape[0]), (0, 0)))
    wr2 = jnp.pad(Wr2, ((0, cp1 - Wr2.shape[0]), (0, 0)))
    xl2, xr2 = _relu_mm2(h1[:N_NODES], b1p, wl2, wr2)
    h2 = _sc_edge_kernel(cp2)(xl2, xr2, edat, aw, offs, We2[0], att2)

    return _bias_sigmoid(h2[:N_NODES], b2)
